# baseline (speedup 1.0000x reference)
import functools

import jax
import jax.numpy as jnp
from jax import lax
from jax.experimental import pallas as pl
from jax.experimental.pallas import tpu as pltpu
from jax.experimental.pallas import tpu_sc as plsc

_CAPF = 2.0
_NC = 2
_NS = 16
_LW = 16
_BLK = 128


def _router_body(T, E1, E2, C1, C2, R, NBMAX, x_ref, wg_ref, slot_ref, g_ref,
                 srow_ref, rrow_ref, b2p_ref):
    NP = E1 * E2
    NROWS = (NBMAX + 1) * _BLK
    logits = jnp.dot(x_ref[...], wg_ref[...],
                     preferred_element_type=jnp.float32)
    iiE1 = jax.lax.broadcasted_iota(jnp.int32, (R, E1), 1)
    iiNP = jax.lax.broadcasted_iota(jnp.int32, (R, NP), 1)
    rr = jax.lax.broadcasted_iota(jnp.int32, (R, R), 0)
    cc = jax.lax.broadcasted_iota(jnp.int32, (R, R), 1)
    Ltri = (rr >= cc).astype(jnp.float32)

    cnt1 = jnp.zeros((1, E1), jnp.float32)
    cnt2 = jnp.zeros((1, NP), jnp.float32)
    chunks = []
    for c in range(T // R):
        lg = logits[c * R:(c + 1) * R, :]
        l1 = lg[:, 0:E1]
        m1 = jnp.max(l1, axis=1, keepdims=True)
        s1 = jnp.sum(jnp.exp(l1 - m1), axis=1, keepdims=True)
        gate1 = 1.0 / s1
        e1 = jnp.min(jnp.where(l1 >= m1, iiE1, E1), axis=1, keepdims=True)
        e2 = jnp.zeros((R, 1), jnp.int32)
        gate2 = jnp.zeros((R, 1), jnp.float32)
        for b in range(E1):
            l2 = lg[:, E1 + E2 * b:E1 + E2 * (b + 1)]
            m2 = jnp.max(l2, axis=1, keepdims=True)
            s2 = jnp.sum(jnp.exp(l2 - m2), axis=1, keepdims=True)
            e2b = jnp.min(jnp.where(l2 >= m2, iiE1, E2), axis=1, keepdims=True)
            sel = e1 == b
            e2 = jnp.where(sel, e2b, e2)
            gate2 = jnp.where(sel, 1.0 / s2, gate2)
        oh1 = (iiE1 == e1).astype(jnp.float32)
        inc1 = jnp.dot(Ltri, oh1,
                       preferred_element_type=jnp.float32) + cnt1
        pos1 = jnp.sum(inc1 * oh1, axis=1, keepdims=True) - 1.0
        keep1 = pos1 < C1
        pairc = e1 * E2 + e2
        ohpk = ((iiNP == pairc) & keep1).astype(jnp.float32)
        inc2 = jnp.dot(Ltri, ohpk,
                       preferred_element_type=jnp.float32) + cnt2
        pos2 = jnp.sum(inc2 * ohpk, axis=1, keepdims=True) - 1.0
        keep = (pos2 < C2) & keep1 & (pos2 >= 0.0)
        cnt1 = cnt1 + jnp.sum(oh1, axis=0, keepdims=True)
        cnt2 = cnt2 + jnp.sum(ohpk, axis=0, keepdims=True)
        g = jnp.where(keep, gate1 * gate2, 0.0)
        chunks.append((pairc, pos2, keep, g))
    cntk = jnp.minimum(cnt2, float(C2))
    nblk = jnp.floor((cntk + (_BLK - 1)) / _BLK)
    nrows = nblk * _BLK
    qq = jax.lax.broadcasted_iota(jnp.int32, (NP, NP), 0)
    pp = jax.lax.broadcasted_iota(jnp.int32, (NP, NP), 1)
    Ustrict = (qq < pp).astype(jnp.float32)
    srow = jnp.dot(nrows, Ustrict, preferred_element_type=jnp.float32,
                   precision=jax.lax.Precision.HIGHEST)
    sblk = srow / float(_BLK)
    totblk = jnp.sum(nblk, axis=1, keepdims=True)
    jb = jax.lax.broadcasted_iota(jnp.int32, (64, 1), 0).astype(jnp.float32)
    ge = (jb >= sblk).astype(jnp.float32)
    pidx = jnp.sum(ge, axis=1, keepdims=True) - 1.0
    b2p = jnp.where(jb < totblk, pidx, float(NP))
    srow_ref[...] = srow.astype(jnp.int32)
    rrow_ref[...] = nrows.astype(jnp.int32)
    b2p_ref[...] = b2p.astype(jnp.int32)
    srow_i = srow.astype(jnp.int32)
    for c, (pairc, pos2, keep, g) in enumerate(chunks):
        rows = slice(c * R, (c + 1) * R)
        ohp = iiNP == pairc
        stok = jnp.sum(jnp.where(ohp, srow_i, 0), axis=1, keepdims=True)
        slot = jnp.where(keep, stok + pos2.astype(jnp.int32), NROWS - 1)
        slot_ref[rows, :] = slot
        g_ref[rows, :] = jnp.broadcast_to(g, (R, 128))


def _run_router(xt, wg, T, E1, E2, C1, C2, NBMAX):
    NP = E1 * E2
    R = min(256, T)
    body = functools.partial(_router_body, T, E1, E2, C1, C2, R, NBMAX)
    return pl.pallas_call(
        body,
        out_shape=(
            jax.ShapeDtypeStruct((T, 1), jnp.int32),
            jax.ShapeDtypeStruct((T, 128), jnp.float32),
            jax.ShapeDtypeStruct((1, NP), jnp.int32),
            jax.ShapeDtypeStruct((1, NP), jnp.int32),
            jax.ShapeDtypeStruct((64, 1), jnp.int32),
        ),
    )(xt, wg)


def _dispatch_body(T, H, slot_hbm, x_hbm, xc_hbm, idx_v, rows_v, sem):
    wid = lax.axis_index("s") * _NC + lax.axis_index("c")
    per = T // (_NC * _NS)
    base = pl.multiple_of(wid * per, 8)
    pltpu.sync_copy(slot_hbm.at[pl.ds(base, per)], idx_v)
    pltpu.sync_copy(x_hbm.at[pl.ds(base, per)], rows_v)
    pltpu.async_copy(rows_v, xc_hbm.at[idx_v], sem).wait()


def _run_dispatch(slot_flat, xt, T, H, NROWS):
    per = T // (_NC * _NS)
    mesh = plsc.VectorSubcoreMesh(core_axis_name="c", subcore_axis_name="s")
    body = functools.partial(_dispatch_body, T, H)
    return pl.kernel(
        body,
        out_type=jax.ShapeDtypeStruct((NROWS, H), jnp.float32),
        mesh=mesh,
        compiler_params=pltpu.CompilerParams(needs_layout_passes=False),
        scratch_types=[
            pltpu.VMEM((per,), jnp.int32),
            pltpu.VMEM((per, H), jnp.float32),
            pltpu.SemaphoreType.DMA,
        ],
    )(slot_flat, xt)


def _cffn_body(NP, b2p_sm, xc_ref, w1_ref, b1_ref, w2_ref, b2_ref, yc_ref):
    i = pl.program_id(0)
    p_raw = b2p_sm[i]

    @pl.when(p_raw < NP)
    def _():
        h = jax.nn.gelu(
            jnp.dot(xc_ref[...], w1_ref[0],
                    preferred_element_type=jnp.float32) + b1_ref[0])
        yc_ref[...] = (jnp.dot(h, w2_ref[0],
                               preferred_element_type=jnp.float32)
                       + b2_ref[0])


def _run_cffn(b2p, xc, W1r, b1r, W2r, b2r, H, F, NP, NBMAX, NROWS):
    body = functools.partial(_cffn_body, NP)

    def wmap(i, b2p_ref):
        return (jnp.minimum(b2p_ref[i], NP - 1), 0, 0)

    grid_spec = pltpu.PrefetchScalarGridSpec(
        num_scalar_prefetch=1,
        grid=(NBMAX,),
        in_specs=[
            pl.BlockSpec((_BLK, H), lambda i, b: (i, 0)),
            pl.BlockSpec((1, H, F), wmap),
            pl.BlockSpec((1, 1, F), wmap),
            pl.BlockSpec((1, F, H), wmap),
            pl.BlockSpec((1, 1, H), wmap),
        ],
        out_specs=pl.BlockSpec((_BLK, H), lambda i, b: (i, 0)),
    )
    return pl.pallas_call(
        body,
        grid_spec=grid_spec,
        out_shape=jax.ShapeDtypeStruct((NROWS, H), jnp.float32),
    )(b2p, xc, W1r, b1r, W2r, b2r)


def _combine_body(T, H, NROWS, slot_hbm, yc_hbm, y_hbm, idx_v, rows_v, sem):
    wid = lax.axis_index("s") * _NC + lax.axis_index("c")
    per = T // (_NC * _NS)
    base = pl.multiple_of(wid * per, 8)
    pltpu.sync_copy(slot_hbm.at[pl.ds(base, per)], idx_v)
    pltpu.async_copy(yc_hbm.at[idx_v], rows_v, sem).wait()
    pltpu.sync_copy(rows_v, y_hbm.at[pl.ds(base, per)])


def _run_combine(slot_flat, yc, T, H, NROWS):
    per = T // (_NC * _NS)
    mesh = plsc.VectorSubcoreMesh(core_axis_name="c", subcore_axis_name="s")
    body = functools.partial(_combine_body, T, H, NROWS)
    return pl.kernel(
        body,
        out_type=jax.ShapeDtypeStruct((T, H), jnp.float32),
        mesh=mesh,
        compiler_params=pltpu.CompilerParams(needs_layout_passes=False),
        scratch_types=[
            pltpu.VMEM((per,), jnp.int32),
            pltpu.VMEM((per, H), jnp.float32),
            pltpu.SemaphoreType.DMA,
        ],
    )(slot_flat, yc)


def _dense_body(y_ref, g_ref, wd_ref, bd_ref, o_ref):
    gcol = g_ref[:, 0:1]
    ym = jnp.where(gcol > 0.0, y_ref[...], 0.0) * gcol
    o_ref[...] = (jnp.dot(ym, wd_ref[...],
                          preferred_element_type=jnp.float32) + bd_ref[...])


def _run_dense(y, g_b, Wd, bd2, T, H):
    R = min(256, T)
    return pl.pallas_call(
        _dense_body,
        grid=(T // R,),
        in_specs=[
            pl.BlockSpec((R, H), lambda i: (i, 0)),
            pl.BlockSpec((R, 128), lambda i: (i, 0)),
            pl.BlockSpec((H, H), lambda i: (0, 0)),
            pl.BlockSpec((1, H), lambda i: (0, 0)),
        ],
        out_specs=pl.BlockSpec((R, H), lambda i: (i, 0)),
        out_shape=jax.ShapeDtypeStruct((T, H), jnp.float32),
    )(y, g_b, Wd, bd2)


def kernel(x, Wg1, Wg2, W1, b1, W2, b2, Wd, bd):
    B, S, H = x.shape
    T = B * S
    E1 = Wg1.shape[1]
    E2 = Wg2.shape[2]
    F = W1.shape[3]
    NP = E1 * E2
    C1 = int(_CAPF * T / E1)
    C2 = int(_CAPF * C1 / E2)
    NBMAX = min(NP * C2, T + (NP - 1) * _BLK) // _BLK
    NROWS = (NBMAX + 1) * _BLK

    xt = x.reshape(T, H)
    wg2m = jnp.transpose(Wg2, (1, 0, 2)).reshape(H, NP)
    pad = (-(E1 + NP)) % 128 if (E1 + NP) > 32 else 32 - (E1 + NP)
    wg = jnp.concatenate(
        [Wg1, wg2m, jnp.zeros((H, pad), jnp.float32)], axis=1)

    slot, g_b, srow, rrow, b2p = _run_router(xt, wg, T, E1, E2, C1, C2, NBMAX)
    slot_flat = slot.reshape(T)

    xc = _run_dispatch(slot_flat, xt, T, H, NROWS)

    yc = _run_cffn(b2p.reshape(64), xc, W1.reshape(NP, H, F),
                   b1.reshape(NP, 1, F), W2.reshape(NP, F, H),
                   b2.reshape(NP, 1, H), H, F, NP, NBMAX, NROWS)

    y = _run_combine(slot_flat, yc, T, H, NROWS)

    out = _run_dense(y, g_b, Wd, bd.reshape(1, H), T, H)
    return out.reshape(B, S, H)


# speedup vs baseline: 1.0185x; 1.0185x over previous
import functools

import jax
import jax.numpy as jnp
from jax import lax
from jax.experimental import pallas as pl
from jax.experimental.pallas import tpu as pltpu
from jax.experimental.pallas import tpu_sc as plsc

_CAPF = 2.0
_NC = 2
_NS = 16
_LW = 16
_BLK = 128


def _argmax4(c0, c1, c2, c3):
    m = jnp.maximum(jnp.maximum(c0, c1), jnp.maximum(c2, c3))
    s = (jnp.exp(c0 - m) + jnp.exp(c1 - m)
         + jnp.exp(c2 - m) + jnp.exp(c3 - m))
    e = jnp.where(c0 >= m, 0,
                  jnp.where(c1 >= m, 1, jnp.where(c2 >= m, 2, 3)))
    return e, 1.0 / s


def _router_body(T, E1, E2, C1, C2, R, NBMAX, x_ref, wg_ref, slot_ref, g_ref,
                 srow_ref, rrow_ref, b2p_ref, lg_ref):
    NP = E1 * E2
    NROWS = (NBMAX + 1) * _BLK
    lg_ref[...] = jnp.dot(x_ref[...], wg_ref[...],
                          preferred_element_type=jnp.float32)
    iiE1 = jax.lax.broadcasted_iota(jnp.int32, (R, E1), 1)
    iiNP = jax.lax.broadcasted_iota(jnp.int32, (R, NP), 1)
    rr = jax.lax.broadcasted_iota(jnp.int32, (R, R), 0)
    cc = jax.lax.broadcasted_iota(jnp.int32, (R, R), 1)
    Ltri = (rr >= cc).astype(jnp.float32)

    cnt1 = jnp.zeros((1, E1), jnp.float32)
    cnt2 = jnp.zeros((1, NP), jnp.float32)
    chunks = []
    for c in range(T // R):
        lg = lg_ref[c * R:(c + 1) * R, :]
        col = [lg[:, j:j + 1] for j in range(E1 + NP)]
        e1, gate1 = _argmax4(col[0], col[1], col[2], col[3])
        e2 = jnp.zeros((R, 1), jnp.int32)
        gate2 = jnp.zeros((R, 1), jnp.float32)
        for b in range(E1):
            e2b, g2b = _argmax4(*col[E1 + E2 * b:E1 + E2 * (b + 1)])
            sel = e1 == b
            e2 = jnp.where(sel, e2b, e2)
            gate2 = jnp.where(sel, g2b, gate2)
        oh1 = (iiE1 == e1).astype(jnp.float32)
        inc1 = jnp.dot(Ltri, oh1,
                       preferred_element_type=jnp.float32) + cnt1
        pos1 = jnp.sum(inc1 * oh1, axis=1, keepdims=True) - 1.0
        keep1 = pos1 < C1
        pairc = e1 * E2 + e2
        ohpk = ((iiNP == pairc) & keep1).astype(jnp.float32)
        inc2 = jnp.dot(Ltri, ohpk,
                       preferred_element_type=jnp.float32) + cnt2
        pos2 = jnp.sum(inc2 * ohpk, axis=1, keepdims=True) - 1.0
        keep = (pos2 < C2) & keep1 & (pos2 >= 0.0)
        cnt1 = cnt1 + jnp.sum(oh1, axis=0, keepdims=True)
        cnt2 = cnt2 + jnp.sum(ohpk, axis=0, keepdims=True)
        g = jnp.where(keep, gate1 * gate2, 0.0)
        chunks.append((pairc, pos2, keep, g))
    cntk = jnp.minimum(cnt2, float(C2))
    nblk = jnp.floor((cntk + (_BLK - 1)) / _BLK)
    nrows = nblk * _BLK
    qq = jax.lax.broadcasted_iota(jnp.int32, (NP, NP), 0)
    pp = jax.lax.broadcasted_iota(jnp.int32, (NP, NP), 1)
    Ustrict = (qq < pp).astype(jnp.float32)
    srow = jnp.dot(nrows, Ustrict, preferred_element_type=jnp.float32,
                   precision=jax.lax.Precision.HIGHEST)
    sblk = srow / float(_BLK)
    totblk = jnp.sum(nblk, axis=1, keepdims=True)
    jb = jax.lax.broadcasted_iota(jnp.int32, (64, 1), 0).astype(jnp.float32)
    ge = (jb >= sblk).astype(jnp.float32)
    pidx = jnp.sum(ge, axis=1, keepdims=True) - 1.0
    b2p = jnp.where(jb < totblk, pidx, float(NP))
    srow_ref[...] = srow.astype(jnp.int32)
    rrow_ref[...] = nrows.astype(jnp.int32)
    b2p_ref[...] = b2p.astype(jnp.int32)
    srow_i = srow.astype(jnp.int32)
    for c, (pairc, pos2, keep, g) in enumerate(chunks):
        rows = slice(c * R, (c + 1) * R)
        ohp = iiNP == pairc
        stok = jnp.sum(jnp.where(ohp, srow_i, 0), axis=1, keepdims=True)
        slot = jnp.where(keep, stok + pos2.astype(jnp.int32), NROWS - 1)
        slot_ref[rows, :] = slot
        g_ref[rows, :] = jnp.broadcast_to(g, (R, 128))


def _run_router(xt, wg, T, E1, E2, C1, C2, NBMAX):
    NP = E1 * E2
    R = min(256, T)
    body = functools.partial(_router_body, T, E1, E2, C1, C2, R, NBMAX)
    return pl.pallas_call(
        body,
        out_shape=(
            jax.ShapeDtypeStruct((T, 1), jnp.int32),
            jax.ShapeDtypeStruct((T, 128), jnp.float32),
            jax.ShapeDtypeStruct((1, NP), jnp.int32),
            jax.ShapeDtypeStruct((1, NP), jnp.int32),
            jax.ShapeDtypeStruct((64, 1), jnp.int32),
        ),
        scratch_shapes=[pltpu.VMEM((T, wg.shape[1]), jnp.float32)],
    )(xt, wg)


def _dispatch_body(T, H, slot_hbm, x_hbm, xc_hbm, idx_v, rows_v, sem):
    wid = lax.axis_index("s") * _NC + lax.axis_index("c")
    per = T // (_NC * _NS)
    base = pl.multiple_of(wid * per, 8)
    pltpu.sync_copy(slot_hbm.at[pl.ds(base, per)], idx_v)
    pltpu.sync_copy(x_hbm.at[pl.ds(base, per)], rows_v)
    pltpu.async_copy(rows_v, xc_hbm.at[idx_v], sem).wait()


def _run_dispatch(slot_flat, xt, T, H, NROWS):
    per = T // (_NC * _NS)
    mesh = plsc.VectorSubcoreMesh(core_axis_name="c", subcore_axis_name="s")
    body = functools.partial(_dispatch_body, T, H)
    return pl.kernel(
        body,
        out_type=jax.ShapeDtypeStruct((NROWS, H), jnp.float32),
        mesh=mesh,
        compiler_params=pltpu.CompilerParams(needs_layout_passes=False),
        scratch_types=[
            pltpu.VMEM((per,), jnp.int32),
            pltpu.VMEM((per, H), jnp.float32),
            pltpu.SemaphoreType.DMA,
        ],
    )(slot_flat, xt)


def _cffn_body(NP, b2p_sm, xc_ref, w1_ref, b1_ref, w2_ref, b2_ref, yc_ref):
    i = pl.program_id(0)
    p_raw = b2p_sm[i]

    @pl.when(p_raw < NP)
    def _():
        h = jax.nn.gelu(
            jnp.dot(xc_ref[...], w1_ref[0],
                    preferred_element_type=jnp.float32) + b1_ref[0])
        yc_ref[...] = (jnp.dot(h, w2_ref[0],
                               preferred_element_type=jnp.float32)
                       + b2_ref[0])


def _run_cffn(b2p, xc, W1r, b1r, W2r, b2r, H, F, NP, NBMAX, NROWS):
    body = functools.partial(_cffn_body, NP)

    def wmap(i, b2p_ref):
        return (jnp.minimum(b2p_ref[i], NP - 1), 0, 0)

    grid_spec = pltpu.PrefetchScalarGridSpec(
        num_scalar_prefetch=1,
        grid=(NBMAX,),
        in_specs=[
            pl.BlockSpec((_BLK, H), lambda i, b: (i, 0)),
            pl.BlockSpec((1, H, F), wmap),
            pl.BlockSpec((1, 1, F), wmap),
            pl.BlockSpec((1, F, H), wmap),
            pl.BlockSpec((1, 1, H), wmap),
        ],
        out_specs=pl.BlockSpec((_BLK, H), lambda i, b: (i, 0)),
    )
    return pl.pallas_call(
        body,
        grid_spec=grid_spec,
        out_shape=jax.ShapeDtypeStruct((NROWS, H), jnp.float32),
    )(b2p, xc, W1r, b1r, W2r, b2r)


def _combine_body(T, H, NROWS, slot_hbm, yc_hbm, y_hbm, idx_v, rows_v, sem):
    wid = lax.axis_index("s") * _NC + lax.axis_index("c")
    per = T // (_NC * _NS)
    base = pl.multiple_of(wid * per, 8)
    pltpu.sync_copy(slot_hbm.at[pl.ds(base, per)], idx_v)
    pltpu.async_copy(yc_hbm.at[idx_v], rows_v, sem).wait()
    pltpu.sync_copy(rows_v, y_hbm.at[pl.ds(base, per)])


def _run_combine(slot_flat, yc, T, H, NROWS):
    per = T // (_NC * _NS)
    mesh = plsc.VectorSubcoreMesh(core_axis_name="c", subcore_axis_name="s")
    body = functools.partial(_combine_body, T, H, NROWS)
    return pl.kernel(
        body,
        out_type=jax.ShapeDtypeStruct((T, H), jnp.float32),
        mesh=mesh,
        compiler_params=pltpu.CompilerParams(needs_layout_passes=False),
        scratch_types=[
            pltpu.VMEM((per,), jnp.int32),
            pltpu.VMEM((per, H), jnp.float32),
            pltpu.SemaphoreType.DMA,
        ],
    )(slot_flat, yc)


def _dense_body(y_ref, g_ref, wd_ref, bd_ref, o_ref):
    gcol = g_ref[:, 0:1]
    ym = jnp.where(gcol > 0.0, y_ref[...], 0.0) * gcol
    o_ref[...] = (jnp.dot(ym, wd_ref[...],
                          preferred_element_type=jnp.float32) + bd_ref[...])


def _run_dense(y, g_b, Wd, bd2, T, H):
    R = min(256, T)
    return pl.pallas_call(
        _dense_body,
        grid=(T // R,),
        in_specs=[
            pl.BlockSpec((R, H), lambda i: (i, 0)),
            pl.BlockSpec((R, 128), lambda i: (i, 0)),
            pl.BlockSpec((H, H), lambda i: (0, 0)),
            pl.BlockSpec((1, H), lambda i: (0, 0)),
        ],
        out_specs=pl.BlockSpec((R, H), lambda i: (i, 0)),
        out_shape=jax.ShapeDtypeStruct((T, H), jnp.float32),
    )(y, g_b, Wd, bd2)


def kernel(x, Wg1, Wg2, W1, b1, W2, b2, Wd, bd):
    B, S, H = x.shape
    T = B * S
    E1 = Wg1.shape[1]
    E2 = Wg2.shape[2]
    F = W1.shape[3]
    NP = E1 * E2
    C1 = int(_CAPF * T / E1)
    C2 = int(_CAPF * C1 / E2)
    NBMAX = min(NP * C2, T + (NP - 1) * _BLK) // _BLK
    NROWS = (NBMAX + 1) * _BLK

    xt = x.reshape(T, H)
    wg2m = jnp.transpose(Wg2, (1, 0, 2)).reshape(H, NP)
    pad = (-(E1 + NP)) % 128 if (E1 + NP) > 32 else 32 - (E1 + NP)
    wg = jnp.concatenate(
        [Wg1, wg2m, jnp.zeros((H, pad), jnp.float32)], axis=1)

    slot, g_b, srow, rrow, b2p = _run_router(xt, wg, T, E1, E2, C1, C2, NBMAX)
    slot_flat = slot.reshape(T)

    xc = _run_dispatch(slot_flat, xt, T, H, NROWS)

    yc = _run_cffn(b2p.reshape(64), xc, W1.reshape(NP, H, F),
                   b1.reshape(NP, 1, F), W2.reshape(NP, F, H),
                   b2.reshape(NP, 1, H), H, F, NP, NBMAX, NROWS)

    y = _run_combine(slot_flat, yc, T, H, NROWS)

    out = _run_dense(y, g_b, Wd, bd.reshape(1, H), T, H)
    return out.reshape(B, S, H)


# speedup vs baseline: 1.0280x; 1.0093x over previous
import functools

import jax
import jax.numpy as jnp
from jax import lax
from jax.experimental import pallas as pl
from jax.experimental.pallas import tpu as pltpu
from jax.experimental.pallas import tpu_sc as plsc

_CAPF = 2.0
_NC = 2
_NS = 16
_LW = 16
_BLK = 128


def _argmax4(c0, c1, c2, c3):
    m = jnp.maximum(jnp.maximum(c0, c1), jnp.maximum(c2, c3))
    s = (jnp.exp(c0 - m) + jnp.exp(c1 - m)
         + jnp.exp(c2 - m) + jnp.exp(c3 - m))
    e = jnp.where(c0 >= m, 0,
                  jnp.where(c1 >= m, 1, jnp.where(c2 >= m, 2, 3)))
    return e, 1.0 / s


def _router_body(T, E1, E2, C1, C2, R, NBMAX, NCH, x_ref, wg_ref, slot_ref,
                 g_ref, srow_ref, rrow_ref, b2p_ref, cnts_ref, meta_ref):
    i = pl.program_id(0)
    NP = E1 * E2
    NROWS = (NBMAX + 1) * _BLK
    iiE1 = jax.lax.broadcasted_iota(jnp.int32, (R, E1), 1)
    iiNP = jax.lax.broadcasted_iota(jnp.int32, (R, NP), 1)

    @pl.when(i == 0)
    def _():
        cnts_ref[...] = jnp.zeros_like(cnts_ref)

    @pl.when(i < NCH)
    def _phase_chunk():
        lg = jnp.dot(x_ref[...], wg_ref[...],
                     preferred_element_type=jnp.float32)
        col = [lg[:, j:j + 1] for j in range(E1 + E1 * E2)]
        e1, gate1 = _argmax4(col[0], col[1], col[2], col[3])
        e2 = jnp.zeros((R, 1), jnp.int32)
        gate2 = jnp.zeros((R, 1), jnp.float32)
        for b in range(E1):
            e2b, g2b = _argmax4(*col[E1 + E2 * b:E1 + E2 * (b + 1)])
            sel = e1 == b
            e2 = jnp.where(sel, e2b, e2)
            gate2 = jnp.where(sel, g2b, gate2)
        rr = jax.lax.broadcasted_iota(jnp.int32, (R, R), 0)
        cc = jax.lax.broadcasted_iota(jnp.int32, (R, R), 1)
        Ltri = (rr >= cc).astype(jnp.float32)
        cnt1 = cnts_ref[0:1, 0:E1]
        cnt2 = cnts_ref[1:2, 0:NP]
        oh1 = (iiE1 == e1).astype(jnp.float32)
        inc1 = jnp.dot(Ltri, oh1, preferred_element_type=jnp.float32) + cnt1
        pos1 = jnp.sum(inc1 * oh1, axis=1, keepdims=True) - 1.0
        keep1 = pos1 < C1
        pairc = e1 * E2 + e2
        ohpk = ((iiNP == pairc) & keep1).astype(jnp.float32)
        inc2 = jnp.dot(Ltri, ohpk, preferred_element_type=jnp.float32) + cnt2
        pos2 = jnp.sum(inc2 * ohpk, axis=1, keepdims=True) - 1.0
        keep = (pos2 < C2) & keep1 & (pos2 >= 0.0)
        cnts_ref[0:1, 0:E1] = cnt1 + jnp.sum(oh1, axis=0, keepdims=True)
        cnts_ref[1:2, 0:NP] = cnt2 + jnp.sum(ohpk, axis=0, keepdims=True)
        g = jnp.where(keep, gate1 * gate2, 0.0)
        rows = pl.ds(i * R, R)
        meta_ref[rows, 0:1] = pairc.astype(jnp.float32)
        meta_ref[rows, 1:2] = pos2
        meta_ref[rows, 2:3] = keep.astype(jnp.float32)
        g_ref[...] = jnp.broadcast_to(g, (R, 128))

    @pl.when(i == NCH)
    def _finalize():
        cnt2 = cnts_ref[1:2, 0:NP]
        cntk = jnp.minimum(cnt2, float(C2))
        nblk = jnp.floor((cntk + (_BLK - 1)) / _BLK)
        nrows = nblk * _BLK
        qq = jax.lax.broadcasted_iota(jnp.int32, (NP, NP), 0)
        pp = jax.lax.broadcasted_iota(jnp.int32, (NP, NP), 1)
        Ustrict = (qq < pp).astype(jnp.float32)
        srow = jnp.dot(nrows, Ustrict, preferred_element_type=jnp.float32,
                       precision=jax.lax.Precision.HIGHEST)
        sblk = srow / float(_BLK)
        totblk = jnp.sum(nblk, axis=1, keepdims=True)
        jb = jax.lax.broadcasted_iota(
            jnp.int32, (64, 1), 0).astype(jnp.float32)
        ge = (jb >= sblk).astype(jnp.float32)
        pidx = jnp.sum(ge, axis=1, keepdims=True) - 1.0
        b2p = jnp.where(jb < totblk, pidx, float(NP))
        srow_ref[...] = srow.astype(jnp.int32)
        rrow_ref[...] = nrows.astype(jnp.int32)
        b2p_ref[...] = b2p.astype(jnp.int32)
        srow_i = srow.astype(jnp.int32)
        for c in range(NCH):
            rows = slice(c * R, (c + 1) * R)
            pairc = meta_ref[rows, 0:1].astype(jnp.int32)
            pos2 = meta_ref[rows, 1:2]
            keep = meta_ref[rows, 2:3] > 0.0
            ohp = iiNP == pairc
            stok = jnp.sum(jnp.where(ohp, srow_i, 0), axis=1, keepdims=True)
            slot = jnp.where(keep, stok + pos2.astype(jnp.int32), NROWS - 1)
            slot_ref[rows, :] = slot


def _run_router(xt, wg, T, E1, E2, C1, C2, NBMAX):
    NP = E1 * E2
    R = min(256, T)
    NCH = T // R
    body = functools.partial(_router_body, T, E1, E2, C1, C2, R, NBMAX, NCH)
    return pl.pallas_call(
        body,
        grid=(NCH + 1,),
        in_specs=[
            pl.BlockSpec((R, xt.shape[1]),
                         lambda i: (jnp.minimum(i, NCH - 1), 0)),
            pl.BlockSpec((wg.shape[0], wg.shape[1]), lambda i: (0, 0)),
        ],
        out_specs=(
            pl.BlockSpec((T, 1), lambda i: (0, 0)),
            pl.BlockSpec((R, 128), lambda i: (jnp.minimum(i, NCH - 1), 0)),
            pl.BlockSpec((1, NP), lambda i: (0, 0)),
            pl.BlockSpec((1, NP), lambda i: (0, 0)),
            pl.BlockSpec((64, 1), lambda i: (0, 0)),
        ),
        out_shape=(
            jax.ShapeDtypeStruct((T, 1), jnp.int32),
            jax.ShapeDtypeStruct((T, 128), jnp.float32),
            jax.ShapeDtypeStruct((1, NP), jnp.int32),
            jax.ShapeDtypeStruct((1, NP), jnp.int32),
            jax.ShapeDtypeStruct((64, 1), jnp.int32),
        ),
        scratch_shapes=[pltpu.VMEM((2, NP), jnp.float32),
                        pltpu.VMEM((T, 4), jnp.float32)],
    )(xt, wg)


def _dispatch_body(T, H, slot_hbm, x_hbm, xc_hbm, idx_v, rows_v, sem):
    wid = lax.axis_index("s") * _NC + lax.axis_index("c")
    per = T // (_NC * _NS)
    base = pl.multiple_of(wid * per, 8)
    pltpu.sync_copy(slot_hbm.at[pl.ds(base, per)], idx_v)
    pltpu.sync_copy(x_hbm.at[pl.ds(base, per)], rows_v)
    pltpu.async_copy(rows_v, xc_hbm.at[idx_v], sem).wait()


def _run_dispatch(slot_flat, xt, T, H, NROWS):
    per = T // (_NC * _NS)
    mesh = plsc.VectorSubcoreMesh(core_axis_name="c", subcore_axis_name="s")
    body = functools.partial(_dispatch_body, T, H)
    return pl.kernel(
        body,
        out_type=jax.ShapeDtypeStruct((NROWS, H), jnp.float32),
        mesh=mesh,
        compiler_params=pltpu.CompilerParams(needs_layout_passes=False),
        scratch_types=[
            pltpu.VMEM((per,), jnp.int32),
            pltpu.VMEM((per, H), jnp.float32),
            pltpu.SemaphoreType.DMA,
        ],
    )(slot_flat, xt)


def _cffn_body(NP, b2p_sm, xc_ref, w1_ref, b1_ref, w2_ref, b2_ref, yc_ref):
    i = pl.program_id(0)
    p_raw = b2p_sm[i]

    @pl.when(p_raw < NP)
    def _():
        h = jax.nn.gelu(
            jnp.dot(xc_ref[...], w1_ref[0],
                    preferred_element_type=jnp.float32) + b1_ref[0])
        yc_ref[...] = (jnp.dot(h, w2_ref[0],
                               preferred_element_type=jnp.float32)
                       + b2_ref[0])


def _run_cffn(b2p, xc, W1r, b1r, W2r, b2r, H, F, NP, NBMAX, NROWS):
    body = functools.partial(_cffn_body, NP)

    def wmap(i, b2p_ref):
        return (jnp.minimum(b2p_ref[i], NP - 1), 0, 0)

    grid_spec = pltpu.PrefetchScalarGridSpec(
        num_scalar_prefetch=1,
        grid=(NBMAX,),
        in_specs=[
            pl.BlockSpec((_BLK, H), lambda i, b: (i, 0)),
            pl.BlockSpec((1, H, F), wmap),
            pl.BlockSpec((1, 1, F), wmap),
            pl.BlockSpec((1, F, H), wmap),
            pl.BlockSpec((1, 1, H), wmap),
        ],
        out_specs=pl.BlockSpec((_BLK, H), lambda i, b: (i, 0)),
    )
    return pl.pallas_call(
        body,
        grid_spec=grid_spec,
        out_shape=jax.ShapeDtypeStruct((NROWS, H), jnp.float32),
    )(b2p, xc, W1r, b1r, W2r, b2r)


def _combine_body(T, H, NROWS, slot_hbm, yc_hbm, y_hbm, idx_v, rows_v, sem):
    wid = lax.axis_index("s") * _NC + lax.axis_index("c")
    per = T // (_NC * _NS)
    base = pl.multiple_of(wid * per, 8)
    pltpu.sync_copy(slot_hbm.at[pl.ds(base, per)], idx_v)
    pltpu.async_copy(yc_hbm.at[idx_v], rows_v, sem).wait()
    pltpu.sync_copy(rows_v, y_hbm.at[pl.ds(base, per)])


def _run_combine(slot_flat, yc, T, H, NROWS):
    per = T // (_NC * _NS)
    mesh = plsc.VectorSubcoreMesh(core_axis_name="c", subcore_axis_name="s")
    body = functools.partial(_combine_body, T, H, NROWS)
    return pl.kernel(
        body,
        out_type=jax.ShapeDtypeStruct((T, H), jnp.float32),
        mesh=mesh,
        compiler_params=pltpu.CompilerParams(needs_layout_passes=False),
        scratch_types=[
            pltpu.VMEM((per,), jnp.int32),
            pltpu.VMEM((per, H), jnp.float32),
            pltpu.SemaphoreType.DMA,
        ],
    )(slot_flat, yc)


def _dense_body(y_ref, g_ref, wd_ref, bd_ref, o_ref):
    gcol = g_ref[:, 0:1]
    ym = jnp.where(gcol > 0.0, y_ref[...], 0.0) * gcol
    o_ref[...] = (jnp.dot(ym, wd_ref[...],
                          preferred_element_type=jnp.float32) + bd_ref[...])


def _run_dense(y, g_b, Wd, bd2, T, H):
    R = min(256, T)
    return pl.pallas_call(
        _dense_body,
        grid=(T // R,),
        in_specs=[
            pl.BlockSpec((R, H), lambda i: (i, 0)),
            pl.BlockSpec((R, 128), lambda i: (i, 0)),
            pl.BlockSpec((H, H), lambda i: (0, 0)),
            pl.BlockSpec((1, H), lambda i: (0, 0)),
        ],
        out_specs=pl.BlockSpec((R, H), lambda i: (i, 0)),
        out_shape=jax.ShapeDtypeStruct((T, H), jnp.float32),
    )(y, g_b, Wd, bd2)


def kernel(x, Wg1, Wg2, W1, b1, W2, b2, Wd, bd):
    B, S, H = x.shape
    T = B * S
    E1 = Wg1.shape[1]
    E2 = Wg2.shape[2]
    F = W1.shape[3]
    NP = E1 * E2
    C1 = int(_CAPF * T / E1)
    C2 = int(_CAPF * C1 / E2)
    NBMAX = min(NP * C2, T + (NP - 1) * _BLK) // _BLK
    NROWS = (NBMAX + 1) * _BLK

    xt = x.reshape(T, H)
    wg2m = jnp.transpose(Wg2, (1, 0, 2)).reshape(H, NP)
    pad = (-(E1 + NP)) % 128 if (E1 + NP) > 32 else 32 - (E1 + NP)
    wg = jnp.concatenate(
        [Wg1, wg2m, jnp.zeros((H, pad), jnp.float32)], axis=1)

    slot, g_b, srow, rrow, b2p = _run_router(xt, wg, T, E1, E2, C1, C2, NBMAX)
    slot_flat = slot.reshape(T)

    xc = _run_dispatch(slot_flat, xt, T, H, NROWS)

    yc = _run_cffn(b2p.reshape(64), xc, W1.reshape(NP, H, F),
                   b1.reshape(NP, 1, F), W2.reshape(NP, F, H),
                   b2.reshape(NP, 1, H), H, F, NP, NBMAX, NROWS)

    y = _run_combine(slot_flat, yc, T, H, NROWS)

    out = _run_dense(y, g_b, Wd, bd.reshape(1, H), T, H)
    return out.reshape(B, S, H)


# speedup vs baseline: 1.1070x; 1.0769x over previous
import functools

import jax
import jax.numpy as jnp
from jax import lax
from jax.experimental import pallas as pl
from jax.experimental.pallas import tpu as pltpu
from jax.experimental.pallas import tpu_sc as plsc

_CAPF = 2.0
_NC = 2
_NS = 16
_LW = 16
_BLK = 256


def _argmax4(c0, c1, c2, c3):
    m = jnp.maximum(jnp.maximum(c0, c1), jnp.maximum(c2, c3))
    s = (jnp.exp(c0 - m) + jnp.exp(c1 - m)
         + jnp.exp(c2 - m) + jnp.exp(c3 - m))
    e = jnp.where(c0 >= m, 0,
                  jnp.where(c1 >= m, 1, jnp.where(c2 >= m, 2, 3)))
    return e, 1.0 / s


def _router_body(T, E1, E2, C1, C2, R, NBMAX, NCH, x_ref, wg_ref, slot_ref,
                 g_ref, srow_ref, rrow_ref, b2p_ref, cnts_ref, meta_ref):
    i = pl.program_id(0)
    NP = E1 * E2
    NROWS = (NBMAX + 1) * _BLK
    iiE1 = jax.lax.broadcasted_iota(jnp.int32, (R, E1), 1)
    iiNP = jax.lax.broadcasted_iota(jnp.int32, (R, NP), 1)

    @pl.when(i == 0)
    def _():
        cnts_ref[...] = jnp.zeros_like(cnts_ref)

    @pl.when(i < NCH)
    def _phase_chunk():
        lg = jnp.dot(x_ref[...], wg_ref[...],
                     preferred_element_type=jnp.float32)
        col = [lg[:, j:j + 1] for j in range(E1 + E1 * E2)]
        e1, gate1 = _argmax4(col[0], col[1], col[2], col[3])
        e2 = jnp.zeros((R, 1), jnp.int32)
        gate2 = jnp.zeros((R, 1), jnp.float32)
        for b in range(E1):
            e2b, g2b = _argmax4(*col[E1 + E2 * b:E1 + E2 * (b + 1)])
            sel = e1 == b
            e2 = jnp.where(sel, e2b, e2)
            gate2 = jnp.where(sel, g2b, gate2)
        rr = jax.lax.broadcasted_iota(jnp.int32, (R, R), 0)
        cc = jax.lax.broadcasted_iota(jnp.int32, (R, R), 1)
        Ltri = (rr >= cc).astype(jnp.float32)
        cnt1 = cnts_ref[0:1, 0:E1]
        cnt2 = cnts_ref[1:2, 0:NP]
        oh1 = (iiE1 == e1).astype(jnp.float32)
        inc1 = jnp.dot(Ltri, oh1, preferred_element_type=jnp.float32) + cnt1
        pos1 = jnp.sum(inc1 * oh1, axis=1, keepdims=True) - 1.0
        keep1 = pos1 < C1
        pairc = e1 * E2 + e2
        ohpk = ((iiNP == pairc) & keep1).astype(jnp.float32)
        inc2 = jnp.dot(Ltri, ohpk, preferred_element_type=jnp.float32) + cnt2
        pos2 = jnp.sum(inc2 * ohpk, axis=1, keepdims=True) - 1.0
        keep = (pos2 < C2) & keep1 & (pos2 >= 0.0)
        cnts_ref[0:1, 0:E1] = cnt1 + jnp.sum(oh1, axis=0, keepdims=True)
        cnts_ref[1:2, 0:NP] = cnt2 + jnp.sum(ohpk, axis=0, keepdims=True)
        g = jnp.where(keep, gate1 * gate2, 0.0)
        rows = pl.ds(i * R, R)
        meta_ref[rows, 0:1] = pairc.astype(jnp.float32)
        meta_ref[rows, 1:2] = pos2
        meta_ref[rows, 2:3] = keep.astype(jnp.float32)
        g_ref[...] = jnp.broadcast_to(g, (R, 128))

    @pl.when(i == NCH)
    def _finalize():
        cnt2 = cnts_ref[1:2, 0:NP]
        cntk = jnp.minimum(cnt2, float(C2))
        nblk = jnp.floor((cntk + (_BLK - 1)) / _BLK)
        nrows = nblk * _BLK
        qq = jax.lax.broadcasted_iota(jnp.int32, (NP, NP), 0)
        pp = jax.lax.broadcasted_iota(jnp.int32, (NP, NP), 1)
        Ustrict = (qq < pp).astype(jnp.float32)
        srow = jnp.dot(nrows, Ustrict, preferred_element_type=jnp.float32,
                       precision=jax.lax.Precision.HIGHEST)
        sblk = srow / float(_BLK)
        totblk = jnp.sum(nblk, axis=1, keepdims=True)
        jb = jax.lax.broadcasted_iota(
            jnp.int32, (64, 1), 0).astype(jnp.float32)
        ge = (jb >= sblk).astype(jnp.float32)
        pidx = jnp.sum(ge, axis=1, keepdims=True) - 1.0
        b2p = jnp.where(jb < totblk, pidx, float(NP))
        srow_ref[...] = srow.astype(jnp.int32)
        rrow_ref[...] = nrows.astype(jnp.int32)
        b2p_ref[...] = b2p.astype(jnp.int32)
        srow_i = srow.astype(jnp.int32)
        for c in range(NCH):
            rows = slice(c * R, (c + 1) * R)
            pairc = meta_ref[rows, 0:1].astype(jnp.int32)
            pos2 = meta_ref[rows, 1:2]
            keep = meta_ref[rows, 2:3] > 0.0
            ohp = iiNP == pairc
            stok = jnp.sum(jnp.where(ohp, srow_i, 0), axis=1, keepdims=True)
            slot = jnp.where(keep, stok + pos2.astype(jnp.int32), NROWS - 1)
            slot_ref[rows, :] = slot


def _run_router(xt, wg, T, E1, E2, C1, C2, NBMAX):
    NP = E1 * E2
    R = min(256, T)
    NCH = T // R
    body = functools.partial(_router_body, T, E1, E2, C1, C2, R, NBMAX, NCH)
    return pl.pallas_call(
        body,
        grid=(NCH + 1,),
        in_specs=[
            pl.BlockSpec((R, xt.shape[1]),
                         lambda i: (jnp.minimum(i, NCH - 1), 0)),
            pl.BlockSpec((wg.shape[0], wg.shape[1]), lambda i: (0, 0)),
        ],
        out_specs=(
            pl.BlockSpec((T, 1), lambda i: (0, 0)),
            pl.BlockSpec((R, 128), lambda i: (jnp.minimum(i, NCH - 1), 0)),
            pl.BlockSpec((1, NP), lambda i: (0, 0)),
            pl.BlockSpec((1, NP), lambda i: (0, 0)),
            pl.BlockSpec((64, 1), lambda i: (0, 0)),
        ),
        out_shape=(
            jax.ShapeDtypeStruct((T, 1), jnp.int32),
            jax.ShapeDtypeStruct((T, 128), jnp.float32),
            jax.ShapeDtypeStruct((1, NP), jnp.int32),
            jax.ShapeDtypeStruct((1, NP), jnp.int32),
            jax.ShapeDtypeStruct((64, 1), jnp.int32),
        ),
        scratch_shapes=[pltpu.VMEM((2, NP), jnp.float32),
                        pltpu.VMEM((T, 4), jnp.float32)],
    )(xt, wg)


def _dispatch_body(T, H, slot_hbm, x_hbm, xc_hbm, idx_v, rows_v, sem):
    wid = lax.axis_index("s") * _NC + lax.axis_index("c")
    per = T // (_NC * _NS)
    base = pl.multiple_of(wid * per, 8)
    pltpu.sync_copy(slot_hbm.at[pl.ds(base, per)], idx_v)
    pltpu.sync_copy(x_hbm.at[pl.ds(base, per)], rows_v)
    pltpu.async_copy(rows_v, xc_hbm.at[idx_v], sem).wait()


def _run_dispatch(slot_flat, xt, T, H, NROWS):
    per = T // (_NC * _NS)
    mesh = plsc.VectorSubcoreMesh(core_axis_name="c", subcore_axis_name="s")
    body = functools.partial(_dispatch_body, T, H)
    return pl.kernel(
        body,
        out_type=jax.ShapeDtypeStruct((NROWS, H), jnp.float32),
        mesh=mesh,
        compiler_params=pltpu.CompilerParams(needs_layout_passes=False),
        scratch_types=[
            pltpu.VMEM((per,), jnp.int32),
            pltpu.VMEM((per, H), jnp.float32),
            pltpu.SemaphoreType.DMA,
        ],
    )(slot_flat, xt)


def _cffn_body(NP, b2p_sm, xc_ref, w1_ref, b1_ref, w2_ref, b2_ref, yc_ref):
    i = pl.program_id(0)
    p_raw = b2p_sm[i]

    @pl.when(p_raw < NP)
    def _():
        h = jax.nn.gelu(
            jnp.dot(xc_ref[...], w1_ref[0],
                    preferred_element_type=jnp.float32) + b1_ref[0])
        yc_ref[...] = (jnp.dot(h, w2_ref[0],
                               preferred_element_type=jnp.float32)
                       + b2_ref[0])


def _run_cffn(b2p, xc, W1r, b1r, W2r, b2r, H, F, NP, NBMAX, NROWS):
    body = functools.partial(_cffn_body, NP)

    def wmap(i, b2p_ref):
        return (jnp.minimum(b2p_ref[i], NP - 1), 0, 0)

    grid_spec = pltpu.PrefetchScalarGridSpec(
        num_scalar_prefetch=1,
        grid=(NBMAX,),
        in_specs=[
            pl.BlockSpec((_BLK, H), lambda i, b: (i, 0)),
            pl.BlockSpec((1, H, F), wmap),
            pl.BlockSpec((1, 1, F), wmap),
            pl.BlockSpec((1, F, H), wmap),
            pl.BlockSpec((1, 1, H), wmap),
        ],
        out_specs=pl.BlockSpec((_BLK, H), lambda i, b: (i, 0)),
    )
    return pl.pallas_call(
        body,
        grid_spec=grid_spec,
        out_shape=jax.ShapeDtypeStruct((NROWS, H), jnp.float32),
    )(b2p, xc, W1r, b1r, W2r, b2r)


def _combine_body(T, H, NROWS, slot_hbm, yc_hbm, y_hbm, idx_v, rows_v, sem):
    wid = lax.axis_index("s") * _NC + lax.axis_index("c")
    per = T // (_NC * _NS)
    base = pl.multiple_of(wid * per, 8)
    pltpu.sync_copy(slot_hbm.at[pl.ds(base, per)], idx_v)
    pltpu.async_copy(yc_hbm.at[idx_v], rows_v, sem).wait()
    pltpu.sync_copy(rows_v, y_hbm.at[pl.ds(base, per)])


def _run_combine(slot_flat, yc, T, H, NROWS):
    per = T // (_NC * _NS)
    mesh = plsc.VectorSubcoreMesh(core_axis_name="c", subcore_axis_name="s")
    body = functools.partial(_combine_body, T, H, NROWS)
    return pl.kernel(
        body,
        out_type=jax.ShapeDtypeStruct((T, H), jnp.float32),
        mesh=mesh,
        compiler_params=pltpu.CompilerParams(needs_layout_passes=False),
        scratch_types=[
            pltpu.VMEM((per,), jnp.int32),
            pltpu.VMEM((per, H), jnp.float32),
            pltpu.SemaphoreType.DMA,
        ],
    )(slot_flat, yc)


def _dense_body(y_ref, g_ref, wd_ref, bd_ref, o_ref):
    gcol = g_ref[:, 0:1]
    ym = jnp.where(gcol > 0.0, y_ref[...], 0.0) * gcol
    o_ref[...] = (jnp.dot(ym, wd_ref[...],
                          preferred_element_type=jnp.float32) + bd_ref[...])


def _run_dense(y, g_b, Wd, bd2, T, H):
    R = min(256, T)
    return pl.pallas_call(
        _dense_body,
        grid=(T // R,),
        in_specs=[
            pl.BlockSpec((R, H), lambda i: (i, 0)),
            pl.BlockSpec((R, 128), lambda i: (i, 0)),
            pl.BlockSpec((H, H), lambda i: (0, 0)),
            pl.BlockSpec((1, H), lambda i: (0, 0)),
        ],
        out_specs=pl.BlockSpec((R, H), lambda i: (i, 0)),
        out_shape=jax.ShapeDtypeStruct((T, H), jnp.float32),
    )(y, g_b, Wd, bd2)


def kernel(x, Wg1, Wg2, W1, b1, W2, b2, Wd, bd):
    B, S, H = x.shape
    T = B * S
    E1 = Wg1.shape[1]
    E2 = Wg2.shape[2]
    F = W1.shape[3]
    NP = E1 * E2
    C1 = int(_CAPF * T / E1)
    C2 = int(_CAPF * C1 / E2)
    NBMAX = min(NP * C2, T + (NP - 1) * _BLK) // _BLK
    NROWS = (NBMAX + 1) * _BLK

    xt = x.reshape(T, H)
    wg2m = jnp.transpose(Wg2, (1, 0, 2)).reshape(H, NP)
    pad = (-(E1 + NP)) % 128 if (E1 + NP) > 32 else 32 - (E1 + NP)
    wg = jnp.concatenate(
        [Wg1, wg2m, jnp.zeros((H, pad), jnp.float32)], axis=1)

    slot, g_b, srow, rrow, b2p = _run_router(xt, wg, T, E1, E2, C1, C2, NBMAX)
    slot_flat = slot.reshape(T)

    xc = _run_dispatch(slot_flat, xt, T, H, NROWS)

    yc = _run_cffn(b2p.reshape(64), xc, W1.reshape(NP, H, F),
                   b1.reshape(NP, 1, F), W2.reshape(NP, F, H),
                   b2.reshape(NP, 1, H), H, F, NP, NBMAX, NROWS)

    y = _run_combine(slot_flat, yc, T, H, NROWS)

    out = _run_dense(y, g_b, Wd, bd.reshape(1, H), T, H)
    return out.reshape(B, S, H)


# speedup vs baseline: 1.1110x; 1.0036x over previous
import functools

import jax
import jax.numpy as jnp
from jax import lax
from jax.experimental import pallas as pl
from jax.experimental.pallas import tpu as pltpu
from jax.experimental.pallas import tpu_sc as plsc

_CAPF = 2.0
_NC = 2
_NS = 16
_LW = 16
_BLK = 384


def _argmax4(c0, c1, c2, c3):
    m = jnp.maximum(jnp.maximum(c0, c1), jnp.maximum(c2, c3))
    s = (jnp.exp(c0 - m) + jnp.exp(c1 - m)
         + jnp.exp(c2 - m) + jnp.exp(c3 - m))
    e = jnp.where(c0 >= m, 0,
                  jnp.where(c1 >= m, 1, jnp.where(c2 >= m, 2, 3)))
    return e, 1.0 / s


def _router_body(T, E1, E2, C1, C2, R, NBMAX, NCH, x_ref, wg_ref, slot_ref,
                 g_ref, srow_ref, rrow_ref, b2p_ref, cnts_ref, meta_ref):
    i = pl.program_id(0)
    NP = E1 * E2
    NROWS = (NBMAX + 1) * _BLK
    iiE1 = jax.lax.broadcasted_iota(jnp.int32, (R, E1), 1)
    iiNP = jax.lax.broadcasted_iota(jnp.int32, (R, NP), 1)

    @pl.when(i == 0)
    def _():
        cnts_ref[...] = jnp.zeros_like(cnts_ref)

    @pl.when(i < NCH)
    def _phase_chunk():
        lg = jnp.dot(x_ref[...], wg_ref[...],
                     preferred_element_type=jnp.float32)
        col = [lg[:, j:j + 1] for j in range(E1 + E1 * E2)]
        e1, gate1 = _argmax4(col[0], col[1], col[2], col[3])
        e2 = jnp.zeros((R, 1), jnp.int32)
        gate2 = jnp.zeros((R, 1), jnp.float32)
        for b in range(E1):
            e2b, g2b = _argmax4(*col[E1 + E2 * b:E1 + E2 * (b + 1)])
            sel = e1 == b
            e2 = jnp.where(sel, e2b, e2)
            gate2 = jnp.where(sel, g2b, gate2)
        rr = jax.lax.broadcasted_iota(jnp.int32, (R, R), 0)
        cc = jax.lax.broadcasted_iota(jnp.int32, (R, R), 1)
        Ltri = (rr >= cc).astype(jnp.float32)
        cnt1 = cnts_ref[0:1, 0:E1]
        cnt2 = cnts_ref[1:2, 0:NP]
        oh1 = (iiE1 == e1).astype(jnp.float32)
        inc1 = jnp.dot(Ltri, oh1, preferred_element_type=jnp.float32) + cnt1
        pos1 = jnp.sum(inc1 * oh1, axis=1, keepdims=True) - 1.0
        keep1 = pos1 < C1
        pairc = e1 * E2 + e2
        ohpk = ((iiNP == pairc) & keep1).astype(jnp.float32)
        inc2 = jnp.dot(Ltri, ohpk, preferred_element_type=jnp.float32) + cnt2
        pos2 = jnp.sum(inc2 * ohpk, axis=1, keepdims=True) - 1.0
        keep = (pos2 < C2) & keep1 & (pos2 >= 0.0)
        cnts_ref[0:1, 0:E1] = cnt1 + jnp.sum(oh1, axis=0, keepdims=True)
        cnts_ref[1:2, 0:NP] = cnt2 + jnp.sum(ohpk, axis=0, keepdims=True)
        g = jnp.where(keep, gate1 * gate2, 0.0)
        rows = pl.ds(i * R, R)
        meta_ref[rows, 0:1] = pairc.astype(jnp.float32)
        meta_ref[rows, 1:2] = pos2
        meta_ref[rows, 2:3] = keep.astype(jnp.float32)
        g_ref[...] = jnp.broadcast_to(g, (R, 128))

    @pl.when(i == NCH)
    def _finalize():
        cnt2 = cnts_ref[1:2, 0:NP]
        cntk = jnp.minimum(cnt2, float(C2))
        nblk = jnp.floor((cntk + (_BLK - 1)) / _BLK)
        nrows = nblk * _BLK
        qq = jax.lax.broadcasted_iota(jnp.int32, (NP, NP), 0)
        pp = jax.lax.broadcasted_iota(jnp.int32, (NP, NP), 1)
        Ustrict = (qq < pp).astype(jnp.float32)
        srow = jnp.dot(nrows, Ustrict, preferred_element_type=jnp.float32,
                       precision=jax.lax.Precision.HIGHEST)
        sblk = srow / float(_BLK)
        totblk = jnp.sum(nblk, axis=1, keepdims=True)
        jb = jax.lax.broadcasted_iota(
            jnp.int32, (64, 1), 0).astype(jnp.float32)
        ge = (jb >= sblk).astype(jnp.float32)
        pidx = jnp.sum(ge, axis=1, keepdims=True) - 1.0
        b2p = jnp.where(jb < totblk, pidx, float(NP))
        srow_ref[...] = srow.astype(jnp.int32)
        rrow_ref[...] = nrows.astype(jnp.int32)
        b2p_ref[...] = b2p.astype(jnp.int32)
        srow_i = srow.astype(jnp.int32)
        for c in range(NCH):
            rows = slice(c * R, (c + 1) * R)
            pairc = meta_ref[rows, 0:1].astype(jnp.int32)
            pos2 = meta_ref[rows, 1:2]
            keep = meta_ref[rows, 2:3] > 0.0
            ohp = iiNP == pairc
            stok = jnp.sum(jnp.where(ohp, srow_i, 0), axis=1, keepdims=True)
            slot = jnp.where(keep, stok + pos2.astype(jnp.int32), NROWS - 1)
            slot_ref[rows, :] = slot


def _run_router(xt, wg, T, E1, E2, C1, C2, NBMAX):
    NP = E1 * E2
    R = min(256, T)
    NCH = T // R
    body = functools.partial(_router_body, T, E1, E2, C1, C2, R, NBMAX, NCH)
    return pl.pallas_call(
        body,
        grid=(NCH + 1,),
        in_specs=[
            pl.BlockSpec((R, xt.shape[1]),
                         lambda i: (jnp.minimum(i, NCH - 1), 0)),
            pl.BlockSpec((wg.shape[0], wg.shape[1]), lambda i: (0, 0)),
        ],
        out_specs=(
            pl.BlockSpec((T, 1), lambda i: (0, 0)),
            pl.BlockSpec((R, 128), lambda i: (jnp.minimum(i, NCH - 1), 0)),
            pl.BlockSpec((1, NP), lambda i: (0, 0)),
            pl.BlockSpec((1, NP), lambda i: (0, 0)),
            pl.BlockSpec((64, 1), lambda i: (0, 0)),
        ),
        out_shape=(
            jax.ShapeDtypeStruct((T, 1), jnp.int32),
            jax.ShapeDtypeStruct((T, 128), jnp.float32),
            jax.ShapeDtypeStruct((1, NP), jnp.int32),
            jax.ShapeDtypeStruct((1, NP), jnp.int32),
            jax.ShapeDtypeStruct((64, 1), jnp.int32),
        ),
        scratch_shapes=[pltpu.VMEM((2, NP), jnp.float32),
                        pltpu.VMEM((T, 4), jnp.float32)],
    )(xt, wg)


def _dispatch_body(T, H, slot_hbm, x_hbm, xc_hbm, idx_v, rows_v, sem):
    wid = lax.axis_index("s") * _NC + lax.axis_index("c")
    per = T // (_NC * _NS)
    base = pl.multiple_of(wid * per, 8)
    pltpu.sync_copy(slot_hbm.at[pl.ds(base, per)], idx_v)
    pltpu.sync_copy(x_hbm.at[pl.ds(base, per)], rows_v)
    pltpu.async_copy(rows_v, xc_hbm.at[idx_v], sem).wait()


def _run_dispatch(slot_flat, xt, T, H, NROWS):
    per = T // (_NC * _NS)
    mesh = plsc.VectorSubcoreMesh(core_axis_name="c", subcore_axis_name="s")
    body = functools.partial(_dispatch_body, T, H)
    return pl.kernel(
        body,
        out_type=jax.ShapeDtypeStruct((NROWS, H), jnp.float32),
        mesh=mesh,
        compiler_params=pltpu.CompilerParams(needs_layout_passes=False),
        scratch_types=[
            pltpu.VMEM((per,), jnp.int32),
            pltpu.VMEM((per, H), jnp.float32),
            pltpu.SemaphoreType.DMA,
        ],
    )(slot_flat, xt)


def _cffn_body(NP, b2p_sm, xc_ref, w1_ref, b1_ref, w2_ref, b2_ref, yc_ref):
    i = pl.program_id(0)
    p_raw = b2p_sm[i]

    @pl.when(p_raw < NP)
    def _():
        h = jax.nn.gelu(
            jnp.dot(xc_ref[...], w1_ref[0],
                    preferred_element_type=jnp.float32) + b1_ref[0])
        yc_ref[...] = (jnp.dot(h, w2_ref[0],
                               preferred_element_type=jnp.float32)
                       + b2_ref[0])


def _run_cffn(b2p, xc, W1r, b1r, W2r, b2r, H, F, NP, NBMAX, NROWS):
    body = functools.partial(_cffn_body, NP)

    def wmap(i, b2p_ref):
        return (jnp.minimum(b2p_ref[i], NP - 1), 0, 0)

    grid_spec = pltpu.PrefetchScalarGridSpec(
        num_scalar_prefetch=1,
        grid=(NBMAX,),
        in_specs=[
            pl.BlockSpec((_BLK, H), lambda i, b: (i, 0)),
            pl.BlockSpec((1, H, F), wmap),
            pl.BlockSpec((1, 1, F), wmap),
            pl.BlockSpec((1, F, H), wmap),
            pl.BlockSpec((1, 1, H), wmap),
        ],
        out_specs=pl.BlockSpec((_BLK, H), lambda i, b: (i, 0)),
    )
    return pl.pallas_call(
        body,
        grid_spec=grid_spec,
        out_shape=jax.ShapeDtypeStruct((NROWS, H), jnp.float32),
    )(b2p, xc, W1r, b1r, W2r, b2r)


def _combine_body(T, H, NROWS, slot_hbm, yc_hbm, y_hbm, idx_v, rows_v, sem):
    wid = lax.axis_index("s") * _NC + lax.axis_index("c")
    per = T // (_NC * _NS)
    base = pl.multiple_of(wid * per, 8)
    pltpu.sync_copy(slot_hbm.at[pl.ds(base, per)], idx_v)
    pltpu.async_copy(yc_hbm.at[idx_v], rows_v, sem).wait()
    pltpu.sync_copy(rows_v, y_hbm.at[pl.ds(base, per)])


def _run_combine(slot_flat, yc, T, H, NROWS):
    per = T // (_NC * _NS)
    mesh = plsc.VectorSubcoreMesh(core_axis_name="c", subcore_axis_name="s")
    body = functools.partial(_combine_body, T, H, NROWS)
    return pl.kernel(
        body,
        out_type=jax.ShapeDtypeStruct((T, H), jnp.float32),
        mesh=mesh,
        compiler_params=pltpu.CompilerParams(needs_layout_passes=False),
        scratch_types=[
            pltpu.VMEM((per,), jnp.int32),
            pltpu.VMEM((per, H), jnp.float32),
            pltpu.SemaphoreType.DMA,
        ],
    )(slot_flat, yc)


def _dense_body(y_ref, g_ref, wd_ref, bd_ref, o_ref):
    gcol = g_ref[:, 0:1]
    ym = jnp.where(gcol > 0.0, y_ref[...], 0.0) * gcol
    o_ref[...] = (jnp.dot(ym, wd_ref[...],
                          preferred_element_type=jnp.float32) + bd_ref[...])


def _run_dense(y, g_b, Wd, bd2, T, H):
    R = min(256, T)
    return pl.pallas_call(
        _dense_body,
        grid=(T // R,),
        in_specs=[
            pl.BlockSpec((R, H), lambda i: (i, 0)),
            pl.BlockSpec((R, 128), lambda i: (i, 0)),
            pl.BlockSpec((H, H), lambda i: (0, 0)),
            pl.BlockSpec((1, H), lambda i: (0, 0)),
        ],
        out_specs=pl.BlockSpec((R, H), lambda i: (i, 0)),
        out_shape=jax.ShapeDtypeStruct((T, H), jnp.float32),
    )(y, g_b, Wd, bd2)


def kernel(x, Wg1, Wg2, W1, b1, W2, b2, Wd, bd):
    B, S, H = x.shape
    T = B * S
    E1 = Wg1.shape[1]
    E2 = Wg2.shape[2]
    F = W1.shape[3]
    NP = E1 * E2
    C1 = int(_CAPF * T / E1)
    C2 = int(_CAPF * C1 / E2)
    NBMAX = min(NP * C2, T + (NP - 1) * _BLK) // _BLK
    NROWS = (NBMAX + 1) * _BLK

    xt = x.reshape(T, H)
    wg2m = jnp.transpose(Wg2, (1, 0, 2)).reshape(H, NP)
    pad = (-(E1 + NP)) % 128 if (E1 + NP) > 32 else 32 - (E1 + NP)
    wg = jnp.concatenate(
        [Wg1, wg2m, jnp.zeros((H, pad), jnp.float32)], axis=1)

    slot, g_b, srow, rrow, b2p = _run_router(xt, wg, T, E1, E2, C1, C2, NBMAX)
    slot_flat = slot.reshape(T)

    xc = _run_dispatch(slot_flat, xt, T, H, NROWS)

    yc = _run_cffn(b2p.reshape(64), xc, W1.reshape(NP, H, F),
                   b1.reshape(NP, 1, F), W2.reshape(NP, F, H),
                   b2.reshape(NP, 1, H), H, F, NP, NBMAX, NROWS)

    y = _run_combine(slot_flat, yc, T, H, NROWS)

    out = _run_dense(y, g_b, Wd, bd.reshape(1, H), T, H)
    return out.reshape(B, S, H)


# speedup vs baseline: 1.1124x; 1.0012x over previous
import functools

import jax
import jax.numpy as jnp
from jax import lax
from jax.experimental import pallas as pl
from jax.experimental.pallas import tpu as pltpu
from jax.experimental.pallas import tpu_sc as plsc

_CAPF = 2.0
_NC = 2
_NS = 16
_LW = 16
_BLK = 256


def _argmax4(c0, c1, c2, c3):
    m = jnp.maximum(jnp.maximum(c0, c1), jnp.maximum(c2, c3))
    s = (jnp.exp(c0 - m) + jnp.exp(c1 - m)
         + jnp.exp(c2 - m) + jnp.exp(c3 - m))
    e = jnp.where(c0 >= m, 0,
                  jnp.where(c1 >= m, 1, jnp.where(c2 >= m, 2, 3)))
    return e, 1.0 / s


def _router_body(T, E1, E2, C1, C2, R, NBMAX, NCH, x_ref, wg_ref, slot_ref,
                 g_ref, srow_ref, rrow_ref, b2p_ref, cnts_ref, meta_ref):
    i = pl.program_id(0)
    NP = E1 * E2
    NROWS = (NBMAX + 1) * _BLK
    iiE1 = jax.lax.broadcasted_iota(jnp.int32, (R, E1), 1)
    iiNP = jax.lax.broadcasted_iota(jnp.int32, (R, NP), 1)

    @pl.when(i == 0)
    def _():
        cnts_ref[...] = jnp.zeros_like(cnts_ref)

    @pl.when(i < NCH)
    def _phase_chunk():
        lg = jnp.dot(x_ref[...], wg_ref[...],
                     preferred_element_type=jnp.float32)
        col = [lg[:, j:j + 1] for j in range(E1 + E1 * E2)]
        e1, gate1 = _argmax4(col[0], col[1], col[2], col[3])
        e2 = jnp.zeros((R, 1), jnp.int32)
        gate2 = jnp.zeros((R, 1), jnp.float32)
        for b in range(E1):
            e2b, g2b = _argmax4(*col[E1 + E2 * b:E1 + E2 * (b + 1)])
            sel = e1 == b
            e2 = jnp.where(sel, e2b, e2)
            gate2 = jnp.where(sel, g2b, gate2)
        rr = jax.lax.broadcasted_iota(jnp.int32, (R, R), 0)
        cc = jax.lax.broadcasted_iota(jnp.int32, (R, R), 1)
        Ltri = (rr >= cc).astype(jnp.float32)
        cnt1 = cnts_ref[0:1, 0:E1]
        cnt2 = cnts_ref[1:2, 0:NP]
        oh1 = (iiE1 == e1).astype(jnp.float32)
        inc1 = jnp.dot(Ltri, oh1, preferred_element_type=jnp.float32) + cnt1
        pos1 = jnp.sum(inc1 * oh1, axis=1, keepdims=True) - 1.0
        keep1 = pos1 < C1
        pairc = e1 * E2 + e2
        ohpk = ((iiNP == pairc) & keep1).astype(jnp.float32)
        inc2 = jnp.dot(Ltri, ohpk, preferred_element_type=jnp.float32) + cnt2
        pos2 = jnp.sum(inc2 * ohpk, axis=1, keepdims=True) - 1.0
        keep = (pos2 < C2) & keep1 & (pos2 >= 0.0)
        cnts_ref[0:1, 0:E1] = cnt1 + jnp.sum(oh1, axis=0, keepdims=True)
        cnts_ref[1:2, 0:NP] = cnt2 + jnp.sum(ohpk, axis=0, keepdims=True)
        g = jnp.where(keep, gate1 * gate2, 0.0)
        rows = pl.ds(i * R, R)
        meta_ref[rows, 0:1] = pairc.astype(jnp.float32)
        meta_ref[rows, 1:2] = pos2
        meta_ref[rows, 2:3] = keep.astype(jnp.float32)
        g_ref[...] = jnp.broadcast_to(g, (R, 128))

    @pl.when(i == NCH)
    def _finalize():
        cnt2 = cnts_ref[1:2, 0:NP]
        cntk = jnp.minimum(cnt2, float(C2))
        nblk = jnp.floor((cntk + (_BLK - 1)) / _BLK)
        nrows = nblk * _BLK
        qq = jax.lax.broadcasted_iota(jnp.int32, (NP, NP), 0)
        pp = jax.lax.broadcasted_iota(jnp.int32, (NP, NP), 1)
        Ustrict = (qq < pp).astype(jnp.float32)
        srow = jnp.dot(nrows, Ustrict, preferred_element_type=jnp.float32,
                       precision=jax.lax.Precision.HIGHEST)
        sblk = srow / float(_BLK)
        totblk = jnp.sum(nblk, axis=1, keepdims=True)
        jb = jax.lax.broadcasted_iota(
            jnp.int32, (64, 1), 0).astype(jnp.float32)
        ge = (jb >= sblk).astype(jnp.float32)
        pidx = jnp.sum(ge, axis=1, keepdims=True) - 1.0
        b2p = jnp.where(jb < totblk, pidx, float(NP))
        srow_ref[...] = srow.astype(jnp.int32)
        rrow_ref[...] = nrows.astype(jnp.int32)
        b2p_ref[...] = b2p.astype(jnp.int32)
        srow_i = srow.astype(jnp.int32)
        for c in range(NCH):
            rows = slice(c * R, (c + 1) * R)
            pairc = meta_ref[rows, 0:1].astype(jnp.int32)
            pos2 = meta_ref[rows, 1:2]
            keep = meta_ref[rows, 2:3] > 0.0
            ohp = iiNP == pairc
            stok = jnp.sum(jnp.where(ohp, srow_i, 0), axis=1, keepdims=True)
            slot = jnp.where(keep, stok + pos2.astype(jnp.int32), NROWS - 1)
            slot_ref[rows, :] = slot


def _run_router(xt, wg, T, E1, E2, C1, C2, NBMAX):
    NP = E1 * E2
    R = min(256, T)
    NCH = T // R
    body = functools.partial(_router_body, T, E1, E2, C1, C2, R, NBMAX, NCH)
    return pl.pallas_call(
        body,
        grid=(NCH + 1,),
        in_specs=[
            pl.BlockSpec((R, xt.shape[1]),
                         lambda i: (jnp.minimum(i, NCH - 1), 0)),
            pl.BlockSpec((wg.shape[0], wg.shape[1]), lambda i: (0, 0)),
        ],
        out_specs=(
            pl.BlockSpec((T, 1), lambda i: (0, 0)),
            pl.BlockSpec((R, 128), lambda i: (jnp.minimum(i, NCH - 1), 0)),
            pl.BlockSpec((1, NP), lambda i: (0, 0)),
            pl.BlockSpec((1, NP), lambda i: (0, 0)),
            pl.BlockSpec((64, 1), lambda i: (0, 0)),
        ),
        out_shape=(
            jax.ShapeDtypeStruct((T, 1), jnp.int32),
            jax.ShapeDtypeStruct((T, 128), jnp.float32),
            jax.ShapeDtypeStruct((1, NP), jnp.int32),
            jax.ShapeDtypeStruct((1, NP), jnp.int32),
            jax.ShapeDtypeStruct((64, 1), jnp.int32),
        ),
        scratch_shapes=[pltpu.VMEM((2, NP), jnp.float32),
                        pltpu.VMEM((T, 4), jnp.float32)],
    )(xt, wg)


def _dispatch_body(T, H, slot_hbm, x_hbm, xc_hbm, idx_v, rows_v, sem):
    wid = lax.axis_index("s") * _NC + lax.axis_index("c")
    per = T // (_NC * _NS)
    base = pl.multiple_of(wid * per, 8)
    pltpu.sync_copy(slot_hbm.at[pl.ds(base, per)], idx_v)
    pltpu.sync_copy(x_hbm.at[pl.ds(base, per)], rows_v)
    pltpu.async_copy(rows_v, xc_hbm.at[idx_v], sem).wait()


def _run_dispatch(slot_flat, xt, T, H, NROWS):
    per = T // (_NC * _NS)
    mesh = plsc.VectorSubcoreMesh(core_axis_name="c", subcore_axis_name="s")
    body = functools.partial(_dispatch_body, T, H)
    return pl.kernel(
        body,
        out_type=jax.ShapeDtypeStruct((NROWS, H), jnp.float32),
        mesh=mesh,
        compiler_params=pltpu.CompilerParams(needs_layout_passes=False),
        scratch_types=[
            pltpu.VMEM((per,), jnp.int32),
            pltpu.VMEM((per, H), jnp.float32),
            pltpu.SemaphoreType.DMA,
        ],
    )(slot_flat, xt)


def _cffn_body(NP, b2p_sm, xc_ref, w1_ref, b1_ref, w2_ref, b2_ref, yc_ref):
    i = pl.program_id(0)
    p_raw = b2p_sm[i]

    @pl.when(p_raw < NP)
    def _():
        h = jax.nn.gelu(
            jnp.dot(xc_ref[...], w1_ref[0],
                    preferred_element_type=jnp.float32) + b1_ref[0])
        yc_ref[...] = (jnp.dot(h, w2_ref[0],
                               preferred_element_type=jnp.float32)
                       + b2_ref[0])


def _run_cffn(b2p, xc, W1r, b1r, W2r, b2r, H, F, NP, NBMAX, NROWS):
    body = functools.partial(_cffn_body, NP)

    def wmap(i, b2p_ref):
        return (jnp.minimum(b2p_ref[i], NP - 1), 0, 0)

    grid_spec = pltpu.PrefetchScalarGridSpec(
        num_scalar_prefetch=1,
        grid=(NBMAX,),
        in_specs=[
            pl.BlockSpec((_BLK, H), lambda i, b: (i, 0)),
            pl.BlockSpec((1, H, F), wmap),
            pl.BlockSpec((1, 1, F), wmap),
            pl.BlockSpec((1, F, H), wmap),
            pl.BlockSpec((1, 1, H), wmap),
        ],
        out_specs=pl.BlockSpec((_BLK, H), lambda i, b: (i, 0)),
    )
    return pl.pallas_call(
        body,
        grid_spec=grid_spec,
        out_shape=jax.ShapeDtypeStruct((NROWS, H), jnp.float32),
    )(b2p, xc, W1r, b1r, W2r, b2r)


def _combine_body(T, H, NROWS, slot_hbm, yc_hbm, y_hbm, idx_v, rows_v, sem):
    wid = lax.axis_index("s") * _NC + lax.axis_index("c")
    per = T // (_NC * _NS)
    base = pl.multiple_of(wid * per, 8)
    pltpu.sync_copy(slot_hbm.at[pl.ds(base, per)], idx_v)
    pltpu.async_copy(yc_hbm.at[idx_v], rows_v, sem).wait()
    pltpu.sync_copy(rows_v, y_hbm.at[pl.ds(base, per)])


def _run_combine(slot_flat, yc, T, H, NROWS):
    per = T // (_NC * _NS)
    mesh = plsc.VectorSubcoreMesh(core_axis_name="c", subcore_axis_name="s")
    body = functools.partial(_combine_body, T, H, NROWS)
    return pl.kernel(
        body,
        out_type=jax.ShapeDtypeStruct((T, H), jnp.float32),
        mesh=mesh,
        compiler_params=pltpu.CompilerParams(needs_layout_passes=False),
        scratch_types=[
            pltpu.VMEM((per,), jnp.int32),
            pltpu.VMEM((per, H), jnp.float32),
            pltpu.SemaphoreType.DMA,
        ],
    )(slot_flat, yc)


def _dense_body(y_ref, g_ref, wd_ref, bd_ref, o_ref):
    gcol = g_ref[:, 0:1]
    ym = jnp.where(gcol > 0.0, y_ref[...], 0.0) * gcol
    o_ref[...] = (jnp.dot(ym, wd_ref[...],
                          preferred_element_type=jnp.float32) + bd_ref[...])


def _run_dense(y, g_b, Wd, bd2, T, H):
    R = min(256, T)
    return pl.pallas_call(
        _dense_body,
        grid=(T // R,),
        in_specs=[
            pl.BlockSpec((R, H), lambda i: (i, 0)),
            pl.BlockSpec((R, 128), lambda i: (i, 0)),
            pl.BlockSpec((H, H), lambda i: (0, 0)),
            pl.BlockSpec((1, H), lambda i: (0, 0)),
        ],
        out_specs=pl.BlockSpec((R, H), lambda i: (i, 0)),
        out_shape=jax.ShapeDtypeStruct((T, H), jnp.float32),
    )(y, g_b, Wd, bd2)


def kernel(x, Wg1, Wg2, W1, b1, W2, b2, Wd, bd):
    B, S, H = x.shape
    T = B * S
    E1 = Wg1.shape[1]
    E2 = Wg2.shape[2]
    F = W1.shape[3]
    NP = E1 * E2
    C1 = int(_CAPF * T / E1)
    C2 = int(_CAPF * C1 / E2)
    NBMAX = min(NP * C2, T + (NP - 1) * _BLK) // _BLK
    NROWS = (NBMAX + 1) * _BLK

    xt = x.reshape(T, H)
    wg2m = jnp.transpose(Wg2, (1, 0, 2)).reshape(H, NP)
    pad = (-(E1 + NP)) % 128 if (E1 + NP) > 32 else 32 - (E1 + NP)
    wg = jnp.concatenate(
        [Wg1, wg2m, jnp.zeros((H, pad), jnp.float32)], axis=1)

    slot, g_b, srow, rrow, b2p = _run_router(xt, wg, T, E1, E2, C1, C2, NBMAX)
    slot_flat = slot.reshape(T)

    xc = _run_dispatch(slot_flat, xt, T, H, NROWS)

    yc = _run_cffn(b2p.reshape(64), xc, W1.reshape(NP, H, F),
                   b1.reshape(NP, 1, F), W2.reshape(NP, F, H),
                   b2.reshape(NP, 1, H), H, F, NP, NBMAX, NROWS)

    y = _run_combine(slot_flat, yc, T, H, NROWS)

    out = _run_dense(y, g_b, Wd, bd.reshape(1, H), T, H)
    return out.reshape(B, S, H)
